# baseline (speedup 1.0000x reference)
import functools

import jax
import jax.numpy as jnp
from jax import lax
from jax.experimental import pallas as pl
from jax.experimental.pallas import tpu as pltpu
from jax.experimental.pallas import tpu_sc as plsc

N_LANE = 16
D = 128
ND = D // N_LANE
WINDOW = 3
SLOT = 8


def _sc_scores(node_embed, context_embed, wpair, npair, B, L, KN):
    PL = wpair.shape[0] * 2 // B
    PK = npair.shape[0] * 2 // B
    K = KN // L
    LP = L + (L % 2)
    SW = LP * SLOT
    NP = B // 2
    info = plsc.get_sparse_core_info()
    NW = info.num_cores * info.num_subcores
    WPP = NP // NW

    mesh = plsc.VectorSubcoreMesh(core_axis_name="c", subcore_axis_name="s")

    @functools.partial(
        pl.kernel,
        out_type=jax.ShapeDtypeStruct((NP, 2 * SW), jnp.float32),
        mesh=mesh,
        scratch_types=[
            pltpu.VMEM((WPP * PL,), jnp.int32),
            pltpu.VMEM((WPP * PK,), jnp.int32),
            pltpu.VMEM((2, 2 * L, D), jnp.float32),
            pltpu.VMEM((2, 2 * L, D), jnp.float32),
            pltpu.VMEM((2, 2 * KN, D), jnp.float32),
            pltpu.VMEM((2, 2 * SW), jnp.float32),
            pltpu.SemaphoreType.DMA,
            pltpu.SemaphoreType.DMA,
            pltpu.SemaphoreType.DMA,
            pltpu.SemaphoreType.DMA,
            pltpu.SemaphoreType.DMA,
            pltpu.SemaphoreType.DMA,
            pltpu.SemaphoreType.DMA,
            pltpu.SemaphoreType.DMA,
        ],
    )
    def k(ne_hbm, ce_hbm, walks_hbm, neg_hbm, out_hbm,
          widx, nidx, eu, cv, nv, sb, g0a, g0b, g0c, g1a, g1b, g1c, o0, o1):
        wid = lax.axis_index("s") * info.num_cores + lax.axis_index("c")
        base = wid * WPP
        gsem = ((g0a, g0b, g0c), (g1a, g1b, g1c))
        osem = (o0, o1)
        pltpu.sync_copy(walks_hbm.at[pl.ds(base * PL, WPP * PL)], widx)
        pltpu.sync_copy(neg_hbm.at[pl.ds(base * PK, WPP * PK)], nidx)

        lane = lax.iota(jnp.int32, N_LANE)
        onehot = [lane == i for i in range(N_LANE)]

        def start_gathers(p, slot):
            return (
                pltpu.async_copy(ne_hbm.at[widx.at[pl.ds(p * PL, 2 * L)]],
                                 eu.at[slot], gsem[slot][0]),
                pltpu.async_copy(ce_hbm.at[widx.at[pl.ds(p * PL, 2 * L)]],
                                 cv.at[slot], gsem[slot][1]),
                pltpu.async_copy(ce_hbm.at[nidx.at[pl.ds(p * PK, 2 * KN)]],
                                 nv.at[slot], gsem[slot][2]),
            )

        def load_row(vref, slot, row):
            out = []
            for j in range(ND // 2):
                a = vref[slot, row, pl.ds(2 * j * N_LANE, N_LANE)]
                b = vref[slot, row, pl.ds((2 * j + 1) * N_LANE, N_LANE)]
                out.append((a, b))
            return out

        def compute_scores(slot, sw):
            def dotv(vref, row, u):
                v = load_row(vref, slot, row)
                acc = u[0][0] * v[0][0] + u[0][1] * v[0][1]
                for j in range(1, ND // 2):
                    acc = acc + u[j][0] * v[j][0] + u[j][1] * v[j][1]
                for sh in (8, 4, 2, 1):
                    acc = acc + acc.at[lane ^ sh].get(
                        mode="promise_in_bounds", unique_indices=True)
                return acc

            def l2_body(l2, carry2):
                sv = jnp.zeros((N_LANE,), jnp.float32)
                for dl in range(2):
                    l = l2 * 2 + dl
                    lc = jnp.minimum(l, L - 1)
                    u = load_row(eu, slot, sw * L + lc)
                    for off in range(1, WINDOW + 1):
                        r = jnp.minimum(lc + off, L - 1)
                        tot = dotv(cv, sw * L + r, u)
                        sv = jnp.where(onehot[dl * SLOT + off - 1], tot, sv)
                    for kk in range(K):
                        tot = dotv(nv, sw * KN + lc * K + kk, u)
                        sv = jnp.where(onehot[dl * SLOT + WINDOW + kk], tot, sv)
                sb[slot, pl.ds(sw * SW + l2 * N_LANE, N_LANE)] = sv
                return carry2

            lax.fori_loop(0, LP // 2, l2_body, 0)

        for h in start_gathers(0, 0):
            h.wait()

        def body(i, carry):
            for dl in range(2):
                p = 2 * i + dl
                slot = dl
                hs = start_gathers(jnp.minimum(p + 1, WPP - 1), 1 - slot)
                @pl.when(i > 0)
                def _():
                    pltpu.make_async_copy(
                        sb.at[slot], out_hbm.at[base + p - 2],
                        osem[slot]).wait()
                for sw in range(2):
                    compute_scores(slot, sw)
                pltpu.async_copy(sb.at[slot], out_hbm.at[base + p],
                                 osem[slot])
                for h in hs:
                    h.wait()
            return carry

        lax.fori_loop(0, WPP // 2, body, 0)
        for slot in range(2):
            pltpu.make_async_copy(
                sb.at[slot], out_hbm.at[base + WPP - 2 + slot],
                osem[slot]).wait()

    return k(node_embed, context_embed, wpair, npair)


def _tc_loss(scores, L):
    B, SW = scores.shape

    def body(s_ref, o_ref):
        s = s_ref[...]
        col = lax.broadcasted_iota(jnp.int32, s.shape, 1)
        l = col // SLOT
        slot = col % SLOT
        is_pos = slot < WINDOW
        valid = (is_pos & ((l + slot + 1) < L)) | (~is_pos & (l < L))
        t = jnp.where(is_pos, s, -s)
        ls = jnp.minimum(t, 0.0) - jnp.log1p(jnp.exp(-jnp.abs(t)))
        contrib = jnp.where(valid, -ls, 0.0)
        o_ref[0, 0] = jnp.sum(contrib) / B

    return pl.pallas_call(
        body,
        out_shape=jax.ShapeDtypeStruct((1, 1), jnp.float32),
        out_specs=pl.BlockSpec(memory_space=pltpu.SMEM),
    )(scores)


def _pair_flat(x, width):
    b, c = x.shape
    xp = x.reshape(b // 2, 2 * c)
    return jnp.pad(xp, ((0, 0), (0, width - 2 * c))).reshape(-1)


def kernel(node_embed, context_embed, walks, negatives):
    B, L = walks.shape
    K = negatives.shape[-1]
    KN = L * K
    w = jnp.maximum(walks.astype(jnp.int32), 0)
    n = negatives.astype(jnp.int32).reshape(B, KN)
    PL = -(-2 * L // 8) * 8
    PK = -(-2 * KN // 8) * 8
    pairs = _sc_scores(node_embed, context_embed,
                       _pair_flat(w, PL), _pair_flat(n, PK), B, L, KN)
    LP = L + (L % 2)
    scores = pairs.reshape(B, LP * SLOT)
    loss = _tc_loss(scores, L)
    return loss[0, 0]


# speedup vs baseline: 1.4652x; 1.4652x over previous
import functools

import jax
import jax.numpy as jnp
from jax import lax
from jax.experimental import pallas as pl
from jax.experimental.pallas import tpu as pltpu
from jax.experimental.pallas import tpu_sc as plsc

N_LANE = 16
D = 128
DW = D // 2
NDW = DW // N_LANE
WINDOW = 3
SLOT = 8


def _sc_scores(node_embed, context_embed, walks, neg_flat):
    B, L = walks.shape
    KN = neg_flat.shape[1]
    K = KN // L
    LP = L + (L % 2)
    SW = LP * SLOT
    info = plsc.get_sparse_core_info()
    NW = info.num_cores * info.num_subcores
    WPW = B // NW

    mesh = plsc.VectorSubcoreMesh(core_axis_name="c", subcore_axis_name="s")

    @functools.partial(
        pl.kernel,
        out_type=jax.ShapeDtypeStruct((B, SW), jnp.float32),
        mesh=mesh,
        scratch_types=[
            pltpu.VMEM((WPW, L), jnp.int32),
            pltpu.VMEM((WPW, KN), jnp.int32),
            pltpu.VMEM((2, L, D), jnp.float32),
            pltpu.VMEM((2, L, D), jnp.float32),
            pltpu.VMEM((2, KN, D), jnp.float32),
            pltpu.VMEM((2, SW), jnp.float32),
            pltpu.SemaphoreType.DMA,
            pltpu.SemaphoreType.DMA,
            pltpu.SemaphoreType.DMA,
            pltpu.SemaphoreType.DMA,
            pltpu.SemaphoreType.DMA,
            pltpu.SemaphoreType.DMA,
            pltpu.SemaphoreType.DMA,
            pltpu.SemaphoreType.DMA,
        ],
    )
    def k(ne_hbm, ce_hbm, walks_hbm, neg_hbm, out_hbm,
          widx, nidx, eu, cv, nv, sb, g0a, g0b, g0c, g1a, g1b, g1c, o0, o1):
        wid = lax.axis_index("s") * info.num_cores + lax.axis_index("c")
        base = wid * WPW
        gsem = ((g0a, g0b, g0c), (g1a, g1b, g1c))
        osem = (o0, o1)
        pltpu.sync_copy(walks_hbm.at[pl.ds(base, WPW)], widx)
        pltpu.sync_copy(neg_hbm.at[pl.ds(base, WPW)], nidx)

        lane = lax.iota(jnp.int32, N_LANE)
        onehot = [lane == i for i in range(N_LANE)]

        def start_gathers(w, slot):
            return (
                pltpu.async_copy(ne_hbm.at[widx.at[w]], eu.at[slot],
                                 gsem[slot][0]),
                pltpu.async_copy(ce_hbm.at[widx.at[w]], cv.at[slot],
                                 gsem[slot][1]),
                pltpu.async_copy(ce_hbm.at[nidx.at[w]], nv.at[slot],
                                 gsem[slot][2]),
            )

        def unpack_row(vref, slot, row):
            out = []
            for j in range(NDW):
                a = vref[slot, row, pl.ds(2 * j * N_LANE, N_LANE)]
                b = vref[slot, row, pl.ds((2 * j + 1) * N_LANE, N_LANE)]
                out.append((a, b))
            return out

        def compute_scores(slot):
            def dotv(vref, row, u):
                v = unpack_row(vref, slot, row)
                acc = u[0][0] * v[0][0] + u[0][1] * v[0][1]
                for j in range(1, NDW):
                    acc = acc + u[j][0] * v[j][0] + u[j][1] * v[j][1]
                for sh in (8, 4, 2, 1):
                    acc = acc + acc.at[lane ^ sh].get(
                        mode="promise_in_bounds", unique_indices=True)
                return acc

            def l2_body(l2, carry2):
                sv = jnp.zeros((N_LANE,), jnp.float32)
                for dl in range(2):
                    l = l2 * 2 + dl
                    lc = jnp.minimum(l, L - 1)
                    u = unpack_row(eu, slot, lc)
                    for off in range(1, WINDOW + 1):
                        r = jnp.minimum(lc + off, L - 1)
                        tot = dotv(cv, r, u)
                        sv = jnp.where(onehot[dl * SLOT + off - 1], tot, sv)
                    for kk in range(K):
                        tot = dotv(nv, lc * K + kk, u)
                        sv = jnp.where(onehot[dl * SLOT + WINDOW + kk], tot, sv)
                sb[slot, pl.ds(l2 * N_LANE, N_LANE)] = sv
                return carry2

            lax.fori_loop(0, LP // 2, l2_body, 0)

        for h in start_gathers(0, 0):
            h.wait()

        def body(i, carry):
            for dl in range(2):
                w = 2 * i + dl
                slot = dl
                hs = start_gathers(jnp.minimum(w + 1, WPW - 1), 1 - slot)
                @pl.when(i > 0)
                def _():
                    pltpu.make_async_copy(sb.at[slot],
                                          out_hbm.at[base + w - 2],
                                          osem[slot]).wait()
                compute_scores(slot)
                pltpu.async_copy(sb.at[slot], out_hbm.at[base + w], osem[slot])
                for h in hs:
                    h.wait()
            return carry

        lax.fori_loop(0, WPW // 2, body, 0)
        for slot in range(2):
            pltpu.make_async_copy(sb.at[slot],
                                  out_hbm.at[base + WPW - 2 + slot],
                                  osem[slot]).wait()

    return k(node_embed, context_embed, walks, neg_flat)


def _tc_loss(scores, L):
    B, SW = scores.shape

    def body(s_ref, o_ref):
        s = s_ref[...]
        col = lax.broadcasted_iota(jnp.int32, s.shape, 1)
        l = col // SLOT
        slot = col % SLOT
        is_pos = slot < WINDOW
        valid = (is_pos & ((l + slot + 1) < L)) | (~is_pos & (l < L))
        t = jnp.where(is_pos, s, -s)
        ls = jnp.minimum(t, 0.0) - jnp.log1p(jnp.exp(-jnp.abs(t)))
        contrib = jnp.where(valid, -ls, 0.0)
        o_ref[0, 0] = jnp.sum(contrib) / B

    return pl.pallas_call(
        body,
        out_shape=jax.ShapeDtypeStruct((1, 1), jnp.float32),
        out_specs=pl.BlockSpec(memory_space=pltpu.SMEM),
    )(scores)


def kernel(node_embed, context_embed, walks, negatives):
    B, L = walks.shape
    K = negatives.shape[-1]
    w = jnp.maximum(walks.astype(jnp.int32), 0)
    n = negatives.astype(jnp.int32).reshape(B, L * K)
    scores = _sc_scores(node_embed, context_embed, w, n)
    loss = _tc_loss(scores, L)
    return loss[0, 0]
